# Initial kernel scaffold; baseline (speedup 1.0000x reference)
import functools

import jax
import jax.numpy as jnp
from jax import lax
from jax.experimental import pallas as pl
from jax.experimental.pallas import tpu as pltpu
from jax.experimental.pallas import tpu_sc as plsc

_N_EDGES = 320000
_D = 128
_L = 16
_NC = 2
_NS = 16
_NW = _NC * _NS
_EDGES_PER_W = _N_EDGES // _NW
_CHUNK = 128
_NFULL = _EDGES_PER_W // _CHUNK
_TAIL = _EDGES_PER_W - _NFULL * _CHUNK


def _make_sc_kernel():
  mesh = plsc.VectorSubcoreMesh(core_axis_name="c", subcore_axis_name="s")

  @functools.partial(
      pl.kernel,
      out_type=jax.ShapeDtypeStruct((_NW, _L), jnp.float32),
      mesh=mesh,
      scratch_types=[
          pltpu.VMEM((_CHUNK,), jnp.int32),
          pltpu.VMEM((_CHUNK,), jnp.int32),
          pltpu.VMEM((_CHUNK,), jnp.float32),
          pltpu.VMEM((_CHUNK, _D), jnp.float32),
          pltpu.VMEM((_CHUNK, _D), jnp.float32),
          pltpu.VMEM((_TAIL,), jnp.int32),
          pltpu.VMEM((_TAIL,), jnp.int32),
          pltpu.VMEM((_TAIL,), jnp.float32),
          pltpu.VMEM((_TAIL, _D), jnp.float32),
          pltpu.VMEM((_TAIL, _D), jnp.float32),
          pltpu.VMEM((_L,), jnp.float32),
          pltpu.SemaphoreType.DMA,
          pltpu.SemaphoreType.DMA,
      ],
  )
  def energy_kernel(state_hbm, ind_i_hbm, ind_j_hbm, w_hbm, out_hbm,
                    idx_i_v, idx_j_v, w_v, rows_i_v, rows_j_v,
                    idx_i_t, idx_j_t, w_t, rows_i_t, rows_j_t,
                    out_v, sem_i, sem_j):
    wid = lax.axis_index("s") * _NC + lax.axis_index("c")
    base0 = wid * _EDGES_PER_W

    def do_chunk(base, size, idx_i, idx_j, wv, rows_i, rows_j, acc):
      pltpu.sync_copy(ind_i_hbm.at[pl.ds(base, size)], idx_i)
      pltpu.sync_copy(ind_j_hbm.at[pl.ds(base, size)], idx_j)
      pltpu.sync_copy(w_hbm.at[pl.ds(base, size)], wv)
      cp_i = pltpu.async_copy(state_hbm.at[idx_i], rows_i, sem_i)
      cp_j = pltpu.async_copy(state_hbm.at[idx_j], rows_j, sem_j)
      cp_i.wait()
      cp_j.wait()

      @functools.partial(plsc.parallel_loop, 0, size, carry=acc, unroll=2)
      def final_acc(e, a):
        we = jnp.full((_L,), wv[e], jnp.float32)
        t = rows_i[e, pl.ds(0, _L)] * rows_j[e, pl.ds(0, _L)]
        for k in range(1, _D // _L):
          t = t + rows_i[e, pl.ds(k * _L, _L)] * rows_j[e, pl.ds(k * _L, _L)]
        return a + t * we

      return final_acc

    acc0 = jnp.zeros((_L,), jnp.float32)

    def chunk_body(ci, acc):
      return do_chunk(base0 + ci * _CHUNK, _CHUNK,
                      idx_i_v, idx_j_v, w_v, rows_i_v, rows_j_v, acc)

    acc = lax.fori_loop(0, _NFULL, chunk_body, acc0)
    acc = do_chunk(base0 + _NFULL * _CHUNK, _TAIL,
                   idx_i_t, idx_j_t, w_t, rows_i_t, rows_j_t, acc)

    out_v[...] = acc
    pltpu.sync_copy(out_v, out_hbm.at[wid])

  return energy_kernel


_sc_energy = _make_sc_kernel()


def kernel(state, ind, weights):
  partials = _sc_energy(state, ind[:, 0], ind[:, 1], weights)
  return jnp.sum(partials)


# speedup vs baseline: 2.0470x; 2.0470x over previous
import functools

import jax
import jax.numpy as jnp
from jax import lax
from jax.experimental import pallas as pl
from jax.experimental.pallas import tpu as pltpu
from jax.experimental.pallas import tpu_sc as plsc

_N_EDGES = 320000
_D = 128
_L = 16
_NC = 2
_NS = 16
_NW = _NC * _NS
_EDGES_PER_W = _N_EDGES // _NW
_CHUNK = 128
_NFULL = _EDGES_PER_W // _CHUNK
_TAIL = _EDGES_PER_W - _NFULL * _CHUNK


def _make_sc_kernel():
  mesh = plsc.VectorSubcoreMesh(core_axis_name="c", subcore_axis_name="s")

  @functools.partial(
      pl.kernel,
      out_type=jax.ShapeDtypeStruct((_NW, _L), jnp.float32),
      mesh=mesh,
      scratch_types=[
          pltpu.VMEM((_CHUNK,), jnp.int32),
          pltpu.VMEM((_CHUNK,), jnp.int32),
          pltpu.VMEM((_CHUNK,), jnp.float32),
          pltpu.VMEM((_CHUNK, _D), jnp.float32),
          pltpu.VMEM((_CHUNK, _D), jnp.float32),
          pltpu.VMEM((_TAIL,), jnp.int32),
          pltpu.VMEM((_TAIL,), jnp.int32),
          pltpu.VMEM((_TAIL,), jnp.float32),
          pltpu.VMEM((_TAIL, _D), jnp.float32),
          pltpu.VMEM((_TAIL, _D), jnp.float32),
          pltpu.VMEM((_L,), jnp.float32),
          pltpu.SemaphoreType.DMA,
          pltpu.SemaphoreType.DMA,
      ],
  )
  def energy_kernel(state_hbm, ind_i_hbm, ind_j_hbm, w_hbm, out_hbm,
                    idx_i_v, idx_j_v, w_v, rows_i_v, rows_j_v,
                    idx_i_t, idx_j_t, w_t, rows_i_t, rows_j_t,
                    out_v, sem_i, sem_j):
    wid = lax.axis_index("s") * _NC + lax.axis_index("c")
    base0 = wid * _EDGES_PER_W

    def do_chunk(base, size, idx_i, idx_j, wv, rows_i, rows_j, acc):
      pltpu.sync_copy(ind_i_hbm.at[pl.ds(base, size)], idx_i)
      pltpu.sync_copy(ind_j_hbm.at[pl.ds(base, size)], idx_j)
      pltpu.sync_copy(w_hbm.at[pl.ds(base, size)], wv)
      cp_i = pltpu.async_copy(state_hbm.at[idx_i], rows_i, sem_i)
      cp_j = pltpu.async_copy(state_hbm.at[idx_j], rows_j, sem_j)
      cp_i.wait()
      cp_j.wait()

      @plsc.parallel_loop(0, size // _L, carry=acc, unroll=1)
      def final_acc(g, a):
        wvec = wv[pl.ds(g * _L, _L)]
        for l in range(_L):
          e = g * _L + l
          we = jnp.full((_L,), wvec[l], jnp.float32)
          t = rows_i[e, pl.ds(0, _L)] * rows_j[e, pl.ds(0, _L)]
          for k in range(1, _D // _L):
            t = t + rows_i[e, pl.ds(k * _L, _L)] * rows_j[e, pl.ds(k * _L, _L)]
          a = a + t * we
        return a

      return final_acc

    acc0 = jnp.zeros((_L,), jnp.float32)

    def chunk_body(ci, acc):
      return do_chunk(base0 + ci * _CHUNK, _CHUNK,
                      idx_i_v, idx_j_v, w_v, rows_i_v, rows_j_v, acc)

    acc = lax.fori_loop(0, _NFULL, chunk_body, acc0)
    acc = do_chunk(base0 + _NFULL * _CHUNK, _TAIL,
                   idx_i_t, idx_j_t, w_t, rows_i_t, rows_j_t, acc)

    out_v[...] = acc
    pltpu.sync_copy(out_v, out_hbm.at[wid])

  return energy_kernel


_sc_energy = _make_sc_kernel()


def kernel(state, ind, weights):
  partials = _sc_energy(state, ind[:, 0], ind[:, 1], weights)
  return jnp.sum(partials)


# speedup vs baseline: 3.9499x; 1.9296x over previous
import functools

import jax
import jax.numpy as jnp
from jax import lax
from jax.experimental import pallas as pl
from jax.experimental.pallas import tpu as pltpu
from jax.experimental.pallas import tpu_sc as plsc

_N_EDGES = 320000
_D = 128
_L = 16
_NC = 2
_NS = 16
_NW = _NC * _NS
_EPW = _N_EDGES // _NW
_CE = 64
_NFULL = _EPW // _CE
_TAIL = _EPW - _NFULL * _CE


def _make_sc_kernel():
  mesh = plsc.VectorSubcoreMesh(core_axis_name="c", subcore_axis_name="s")

  @functools.partial(
      pl.kernel,
      out_type=jax.ShapeDtypeStruct((_NW, _L), jnp.float32),
      mesh=mesh,
      scratch_types=[
          pltpu.VMEM((2 * _EPW,), jnp.int32),
          pltpu.VMEM((_EPW,), jnp.float32),
          pltpu.VMEM((2, 2 * _CE, _D), jnp.float32),
          pltpu.VMEM((2 * _TAIL, _D), jnp.float32),
          pltpu.VMEM((_L,), jnp.float32),
          pltpu.SemaphoreType.DMA,
          pltpu.SemaphoreType.DMA,
          pltpu.SemaphoreType.DMA,
      ],
  )
  def energy_kernel(state_hbm, ind_hbm, w_hbm, out_hbm,
                    idx_v, w_v, rows_v, rows_t, out_v, sem0, sem1, sem_t):
    wid = lax.axis_index("s") * _NC + lax.axis_index("c")
    base0 = wid * _EPW

    pltpu.sync_copy(ind_hbm.at[pl.ds(2 * base0, 2 * _EPW)], idx_v)
    pltpu.sync_copy(w_hbm.at[pl.ds(base0, _EPW)], w_v)

    sems = (sem0, sem1)

    def start(ci, slot):
      ib = idx_v.at[pl.ds(ci * 2 * _CE, 2 * _CE)]
      pltpu.async_copy(state_hbm.at[ib], rows_v.at[slot], sems[slot])

    def wait(slot):
      pltpu.make_async_copy(
          state_hbm.at[idx_v.at[pl.ds(0, 2 * _CE)]],
          rows_v.at[slot], sems[slot]).wait()

    def compute(rows, wbase, nedges, acc):
      @plsc.parallel_loop(0, nedges, carry=acc, unroll=2)
      def final_acc(e, a):
        wvec = w_v[pl.ds(wbase + (e // _L) * _L, _L)]
        we = jnp.take_along_axis(
            wvec, jnp.full((_L,), e % _L, jnp.int32), axis=0)
        prods = [rows[2 * e, pl.ds(k * _L, _L)] * rows[2 * e + 1, pl.ds(k * _L, _L)]
                 for k in range(_D // _L)]
        while len(prods) > 1:
          prods = [prods[m] + prods[m + 1] for m in range(0, len(prods) - 1, 2)
                  ] + (prods[-1:] if len(prods) % 2 else [])
        return a + prods[0] * we

      return final_acc

    tb = idx_v.at[pl.ds(_NFULL * 2 * _CE, 2 * _TAIL)]
    tail_cp = pltpu.async_copy(state_hbm.at[tb], rows_t, sem_t)

    start(0, 0)
    acc0 = jnp.zeros((_L,), jnp.float32)

    def outer(t, acc):
      ci0 = t * 2
      start(ci0 + 1, 1)
      wait(0)
      acc = compute(rows_v.at[0], ci0 * _CE, _CE, acc)

      @pl.when(ci0 + 2 < _NFULL)
      def _():
        start(ci0 + 2, 0)

      wait(1)
      acc = compute(rows_v.at[1], (ci0 + 1) * _CE, _CE, acc)
      return acc

    acc = lax.fori_loop(0, _NFULL // 2, outer, acc0)

    tail_cp.wait()
    acc = compute(rows_t, _NFULL * _CE, _TAIL, acc)

    out_v[...] = acc
    pltpu.sync_copy(out_v, out_hbm.at[wid])

  return energy_kernel


_sc_energy = _make_sc_kernel()


def kernel(state, ind, weights):
  partials = _sc_energy(state, ind.reshape(-1), weights)
  return jnp.sum(partials)
